# baseline (speedup 1.0000x reference)
import functools

import jax
import jax.numpy as jnp
from jax import lax
from jax.experimental import pallas as pl
from jax.experimental.pallas import tpu as pltpu
from jax.experimental.pallas import tpu_sc as plsc

VOCAB = 30522
HIDDEN = 768
BATCH = 4
SEQ = 2048
EPS = 1e-12

L = 16
NW = 32
SPW = SEQ // NW
CH = 32
NCH = SPW // CH
NJ = HIDDEN // L
NSTEP = NCH * BATCH


def _row_layernorm(rowbuf, posbuf, gbuf, bbuf, r):
    acc_s = [jnp.zeros((L,), jnp.float32) for _ in range(4)]
    acc_q = [jnp.zeros((L,), jnp.float32) for _ in range(4)]
    for j in range(NJ):
        sl = pl.ds(j * L, L)
        x = rowbuf[r, sl] + posbuf[r, sl]
        rowbuf[r, sl] = x
        acc_s[j % 4] = acc_s[j % 4] + x
        acc_q[j % 4] = acc_q[j % 4] + x * x
    s_tot = (acc_s[0] + acc_s[1]) + (acc_s[2] + acc_s[3])
    q_tot = (acc_q[0] + acc_q[1]) + (acc_q[2] + acc_q[3])
    inv_h = jnp.float32(1.0 / HIDDEN)
    mean_v = jnp.broadcast_to(jnp.sum(s_tot), (L,)) * inv_h
    ex2_v = jnp.broadcast_to(jnp.sum(q_tot), (L,)) * inv_h
    var_v = ex2_v - mean_v * mean_v + jnp.float32(EPS)
    iv = lax.bitcast_convert_type(var_v, jnp.int32)
    iv = jnp.int32(0x5F3759DF) - lax.shift_right_arithmetic(iv, jnp.int32(1))
    y = lax.bitcast_convert_type(iv, jnp.float32)
    half_v = var_v * jnp.float32(0.5)
    for _ in range(3):
        y = y * (jnp.float32(1.5) - half_v * y * y)
    a_v = y
    b_v = -(mean_v * y)
    for j in range(NJ):
        sl = pl.ds(j * L, L)
        t = rowbuf[r, sl] * a_v + b_v
        rowbuf[r, sl] = t * gbuf[sl] + bbuf[sl]


def _sc_body(ids_ref, tab_ref, pos_ref, g_ref, b_ref, out_ref,
             idxbuf, rowbuf, posbuf, gbuf, bbuf, gsem):
    w = lax.axis_index("s") * 2 + lax.axis_index("c")
    pltpu.sync_copy(ids_ref.at[w], idxbuf)
    pltpu.sync_copy(g_ref, gbuf)
    pltpu.sync_copy(b_ref, bbuf)
    for sc in range(NCH):
        s0 = w * SPW + sc * CH
        pltpu.sync_copy(pos_ref.at[pl.ds(s0, CH)], posbuf)

        def step(b, _, sc=sc, s0=s0):
            pltpu.async_copy(tab_ref.at[idxbuf.at[sc * BATCH + b]],
                             rowbuf, gsem).wait()

            def row(r, _):
                _row_layernorm(rowbuf, posbuf, gbuf, bbuf, r)
                return 0

            lax.fori_loop(0, CH, row, 0)
            pltpu.sync_copy(rowbuf, out_ref.at[b, pl.ds(s0, CH)])
            return 0

        lax.fori_loop(0, BATCH, step, 0)


@functools.partial(jax.jit, static_argnums=())
def _run(ids_arr, word_embeddings, position_embeddings, gamma, beta):
    mesh = plsc.VectorSubcoreMesh(core_axis_name="c", subcore_axis_name="s")
    f = functools.partial(
        pl.kernel,
        mesh=mesh,
        compiler_params=pltpu.CompilerParams(needs_layout_passes=False),
        out_type=jax.ShapeDtypeStruct((BATCH, SEQ, HIDDEN), jnp.float32),
        scratch_types=[
            pltpu.VMEM((NSTEP, CH), jnp.int32),
            pltpu.VMEM((CH, HIDDEN), jnp.float32),
            pltpu.VMEM((CH, HIDDEN), jnp.float32),
            pltpu.VMEM((HIDDEN,), jnp.float32),
            pltpu.VMEM((HIDDEN,), jnp.float32),
            pltpu.SemaphoreType.DMA,
        ],
    )(_sc_body)
    return f(ids_arr, word_embeddings, position_embeddings, gamma, beta)


def kernel(input_ids, word_embeddings, position_embeddings, gamma, beta):
    ids_arr = (input_ids.astype(jnp.int32)
               .reshape(BATCH, NW, NCH, CH)
               .transpose(1, 2, 0, 3)
               .reshape(NW, NSTEP, CH))
    return _run(ids_arr, word_embeddings, position_embeddings, gamma, beta)
